# Initial kernel scaffold; baseline (speedup 1.0000x reference)
import jax
import jax.numpy as jnp
from jax.experimental import pallas as pl


def kernel(input, adj_indices, adj_values, W_diag1, W_diag2, W1, b1, W2, b2):
    raise NotImplementedError("write your pallas kernel here")


# speedup vs baseline: 1.4780x; 1.4780x over previous
import jax
import jax.numpy as jnp
from jax.experimental import pallas as pl
from jax.experimental.pallas import tpu as pltpu

_N = 10000
_F = 128
_K = 16
_NP = 10240
_BLK = 128


def _simtopk_body(emb_blk_ref, emb_full_ref, vals_ref, idx_ref):
    s = jax.lax.dot_general(
        emb_blk_ref[...], emb_full_ref[...],
        (((1,), (1,)), ((), ())),
        preferred_element_type=jnp.float32,
    )
    col = jax.lax.broadcasted_iota(jnp.int32, s.shape, 1)
    s = jnp.where(col < _N, s, -jnp.inf)
    for k in range(_K):
        m = jnp.max(s, axis=1, keepdims=True)
        cand = jnp.where(s == m, col, _NP)
        am = jnp.min(cand, axis=1, keepdims=True)
        vals_ref[:, k] = m[:, 0]
        idx_ref[:, k] = am[:, 0]
        s = jnp.where(col == am, -jnp.inf, s)


def _sim_topk(emb):
    emb_p = jnp.zeros((_NP, _F), dtype=jnp.float32).at[:_N].set(emb)
    vals, idx = pl.pallas_call(
        _simtopk_body,
        grid=(_NP // _BLK,),
        in_specs=[
            pl.BlockSpec((_BLK, _F), lambda i: (i, 0)),
            pl.BlockSpec((_NP, _F), lambda i: (0, 0)),
        ],
        out_specs=[
            pl.BlockSpec((_BLK, _K), lambda i: (i, 0)),
            pl.BlockSpec((_BLK, _K), lambda i: (i, 0)),
        ],
        out_shape=[
            jax.ShapeDtypeStruct((_NP, _K), jnp.float32),
            jax.ShapeDtypeStruct((_NP, _K), jnp.int32),
        ],
    )(emb_p, emb_p)
    return vals[:_N], idx[:_N]


def _spmm(indices, values, x):
    gathered = jnp.take(x, indices[1], axis=0) * values[:, None]
    return jax.ops.segment_sum(gathered, indices[0], num_segments=_N)


def _normalize_adj(indices, values):
    deg = jax.ops.segment_sum(values, indices[0], num_segments=_N)
    inv_sqrt = 1.0 / (jnp.sqrt(deg) + 1e-10)
    return values * inv_sqrt[indices[0]] * inv_sqrt[indices[1]]


def kernel(input, adj_indices, adj_values, W_diag1, W_diag2, W1, b1, W2, b2):
    norm_vals = _normalize_adj(adj_indices, adj_values)
    h = jnp.tanh(_spmm(adj_indices, norm_vals, input * W_diag1))
    emb = _spmm(adj_indices, norm_vals, h * W_diag2)
    nrm = jnp.linalg.norm(emb, axis=1, keepdims=True)
    emb = emb / jnp.maximum(nrm, 1e-12)
    vals, idx = _sim_topk(emb)
    rows = jnp.repeat(jnp.arange(_N, dtype=jnp.int32), _K)
    inds = jnp.stack([rows, idx.reshape(-1).astype(jnp.int32)])
    inds_sym = jnp.concatenate([inds, jnp.stack([inds[1], inds[0]])], axis=1)
    vals_flat = vals.reshape(-1)
    vals_sym = jnp.concatenate([vals_flat, vals_flat])
    new_inds = jnp.concatenate([adj_indices.astype(jnp.int32), inds_sym], axis=1)
    new_vals = jnp.concatenate([adj_values, vals_sym])
    norm_new = _normalize_adj(new_inds, new_vals)
    h1 = jax.nn.relu(_spmm(new_inds, norm_new, input @ W1 + b1))
    x_out = _spmm(new_inds, norm_new, h1 @ W2 + b2)
    return (x_out, inds_sym, vals_sym, new_inds, new_vals)


# speedup vs baseline: 3.0166x; 2.0410x over previous
import jax
import jax.numpy as jnp
from jax.experimental import pallas as pl
from jax.experimental.pallas import tpu as pltpu

_N = 10000
_F = 128
_K = 16
_NP = 10240
_BLK = 128


def _simtopk_body(emb_blk_ref, emb_full_ref, vals_ref, idx_ref):
    s = jax.lax.dot_general(
        emb_blk_ref[...], emb_full_ref[...],
        (((1,), (1,)), ((), ())),
        preferred_element_type=jnp.float32,
    )
    col = jax.lax.broadcasted_iota(jnp.int32, s.shape, 1)
    s = jnp.where(col < _N, s, -jnp.inf)
    for k in range(_K):
        m = jnp.max(s, axis=1, keepdims=True)
        cand = jnp.where(s == m, col, _NP)
        am = jnp.min(cand, axis=1, keepdims=True)
        vals_ref[:, k] = m[:, 0]
        idx_ref[:, k] = am[:, 0]
        s = jnp.where(col == am, -jnp.inf, s)


def _sim_topk(emb):
    emb_p = jnp.zeros((_NP, _F), dtype=jnp.float32).at[:_N].set(emb)
    vals, idx = pl.pallas_call(
        _simtopk_body,
        grid=(_NP // _BLK,),
        in_specs=[
            pl.BlockSpec((_BLK, _F), lambda i: (i, 0)),
            pl.BlockSpec((_NP, _F), lambda i: (0, 0)),
        ],
        out_specs=[
            pl.BlockSpec((_BLK, _K), lambda i: (i, 0)),
            pl.BlockSpec((_BLK, _K), lambda i: (i, 0)),
        ],
        out_shape=[
            jax.ShapeDtypeStruct((_NP, _K), jnp.float32),
            jax.ShapeDtypeStruct((_NP, _K), jnp.int32),
        ],
    )(emb_p, emb_p)
    return vals[:_N], idx[:_N]


def _spmm(indices, values, x):
    gathered = jnp.take(x, indices[1], axis=0) * values[:, None]
    return jax.ops.segment_sum(gathered, indices[0], num_segments=_N)


def _normalize_adj(indices, values):
    deg = jax.ops.segment_sum(values, indices[0], num_segments=_N)
    inv_sqrt = 1.0 / (jnp.sqrt(deg) + 1e-10)
    return values * inv_sqrt[indices[0]] * inv_sqrt[indices[1]]


def kernel(input, adj_indices, adj_values, W_diag1, W_diag2, W1, b1, W2, b2):
    deg0 = jax.ops.segment_sum(adj_values, adj_indices[0], num_segments=_N)
    inv0 = 1.0 / (jnp.sqrt(deg0) + 1e-10)
    norm_vals = adj_values * inv0[adj_indices[0]] * inv0[adj_indices[1]]
    h = jnp.tanh(_spmm(adj_indices, norm_vals, input * W_diag1))
    emb = _spmm(adj_indices, norm_vals, h * W_diag2)
    nrm = jnp.linalg.norm(emb, axis=1, keepdims=True)
    emb = emb / jnp.maximum(nrm, 1e-12)
    vals, idx = _sim_topk(emb)
    rows = jnp.repeat(jnp.arange(_N, dtype=jnp.int32), _K)
    idx_flat = idx.reshape(-1)
    inds = jnp.stack([rows, idx_flat])
    inds_sym = jnp.concatenate([inds, jnp.stack([inds[1], inds[0]])], axis=1)
    vals_flat = vals.reshape(-1)
    vals_sym = jnp.concatenate([vals_flat, vals_flat])
    new_inds = jnp.concatenate([adj_indices.astype(jnp.int32), inds_sym], axis=1)
    new_vals = jnp.concatenate([adj_values, vals_sym])
    deg_new = (deg0 + jnp.sum(vals, axis=1)
               + jax.ops.segment_sum(vals_flat, idx_flat, num_segments=_N))
    inv = 1.0 / (jnp.sqrt(deg_new) + 1e-10)

    def spmm_new(z):
        zi = z * inv[:, None]
        part_o = jax.ops.segment_sum(
            jnp.take(zi, adj_indices[1], axis=0) * adj_values[:, None],
            adj_indices[0], num_segments=_N)
        zg = jnp.take(zi, idx_flat, axis=0).reshape(_N, _K, -1)
        part_g = jnp.sum(vals[:, :, None] * zg, axis=1)
        part_t = jax.ops.segment_sum(
            (vals[:, :, None] * zi[:, None, :]).reshape(_N * _K, -1),
            idx_flat, num_segments=_N)
        return inv[:, None] * (part_o + part_g + part_t)

    h1 = jax.nn.relu(spmm_new(input @ W1 + b1))
    x_out = spmm_new(h1 @ W2 + b2)
    return (x_out, inds_sym, vals_sym, new_inds, new_vals)
